# baseline (speedup 1.0000x reference)
import functools

import jax
import jax.numpy as jnp
from jax import lax
from jax.experimental import pallas as pl
from jax.experimental.pallas import tpu as pltpu
from jax.experimental.pallas import tpu_sc as plsc

B = 1024
S = 50
C = 26
D = 128
V = 256
LOW = -3.0
HIGH = 3.0

NC = 2
NS = 16
NW = NC * NS
BPW = B // NW

SP = 56
CP = 128
RP = SP * CP
BINS = C * V
CB = 4
NCK = BPW // CB


def _tc_quant(x):
    BM = 128

    def body(x_ref, o_ref):
        v = x_ref[...]
        t = ((v - LOW) / (HIGH - LOW)) * float(V - 1)
        q = jnp.clip(jnp.round(t), 0.0, float(V - 1))
        idx = q.astype(jnp.int32)
        ch = lax.broadcasted_iota(jnp.int32, (BM, S, C), 2)
        o_ref[:, :S, :C] = ch * V + idx

    return pl.pallas_call(
        body,
        grid=(B // BM,),
        in_specs=[pl.BlockSpec((BM, S, C), lambda i: (i, 0, 0))],
        out_specs=pl.BlockSpec((BM, SP, CP), lambda i: (i, 0, 0)),
        out_shape=jax.ShapeDtypeStruct((B, SP, CP), jnp.int32),
    )(x)


def _sc_hist(addr_flat):
    mesh = plsc.VectorSubcoreMesh(core_axis_name="c", subcore_axis_name="s")
    NBUF = 4
    CW = CB * RP

    @functools.partial(
        pl.kernel,
        out_type=jax.ShapeDtypeStruct((B * BINS,), jnp.float32),
        mesh=mesh,
        scratch_types=[
            pltpu.VMEM((2 * CW,), jnp.int32),
            pltpu.VMEM((NBUF * BINS,), jnp.float32),
            pltpu.VMEM_SHARED((BINS,), jnp.float32),
            [pltpu.SemaphoreType.DMA] * 2,
            [pltpu.SemaphoreType.DMA] * NBUF,
            [pltpu.SemaphoreType.DMA] * NBUF,
        ],
        compiler_params=pltpu.CompilerParams(needs_layout_passes=False),
    )
    def hist(a_hbm, cnt_hbm, a_v, bins_v, zsp, isem, osem, zsem):
        sid = lax.axis_index("s")
        wid = sid * NC + lax.axis_index("c")
        base_b = wid * BPW

        lane = lax.iota(jnp.int32, 16)
        ones = jnp.full((16,), 1.0, jnp.float32)
        zeros = jnp.zeros((16,), jnp.float32)
        k1_mask = lane < (C - 16)

        def in_load2(ck, hh):
            pltpu.async_copy(
                a_hbm.at[pl.ds((base_b + ck * CB) * RP, CW)],
                a_v.at[pl.ds(hh * CW, CW)],
                isem[hh],
            )

        in_load2(0, 0)
        in_load2(1, 1)

        def zero_body(z, c2):
            for k in range(8):
                bins_v[pl.ds(z * 128 + k * 16, 16)] = zeros
            return c2

        lax.fori_loop(0, NBUF * BINS // 128, zero_body, 0)

        @pl.when(sid == 0)
        def _publish_zeros():
            pltpu.sync_copy(bins_v.at[pl.ds(0, BINS)], zsp)

        plsc.subcore_barrier()

        def scat(hh, j):
            boff = hh * CW + j * RP
            pbase = j * BINS
            for s in range(S):
                for k in range(2):
                    a = a_v[pl.ds(boff + s * CP + k * 16, 16)] + pbase
                    if k == 0:
                        plsc.addupdate_scatter(bins_v, [a], ones)
                    else:
                        plsc.addupdate_scatter(bins_v, [a], ones, mask=k1_mask)

        def chunk_pair_body(i, carry):
            for hh in range(2):
                ck = i * 2 + hh

                pltpu.make_async_copy(
                    a_hbm.at[pl.ds(0, CW)],
                    a_v.at[pl.ds(hh * CW, CW)],
                    isem[hh],
                ).wait()

                for j in range(NBUF):
                    @pl.when(ck > 0)
                    def _wait_zero():
                        pltpu.make_async_copy(
                            zsp, bins_v.at[pl.ds(j * BINS, BINS)],
                            zsem[j]).wait()

                    scat(hh, j)

                    pltpu.async_copy(
                        bins_v.at[pl.ds(j * BINS, BINS)],
                        cnt_hbm.at[pl.ds(
                            (base_b + ck * CB + j) * BINS, BINS)],
                        osem[j],
                    )

                    r2 = (j + 2) % NBUF

                    @pl.when((ck > 0) | (j >= 2))
                    def _refill():
                        pltpu.make_async_copy(
                            bins_v.at[pl.ds(r2 * BINS, BINS)],
                            cnt_hbm.at[pl.ds(0, BINS)],
                            osem[r2],
                        ).wait()
                        pltpu.async_copy(
                            zsp, bins_v.at[pl.ds(r2 * BINS, BINS)], zsem[r2])

                @pl.when(ck + 2 < NCK)
                def _next_in():
                    in_load2(ck + 2, hh)
            return carry

        lax.fori_loop(0, NCK // 2, chunk_pair_body, 0)

        for j in range(2):
            pltpu.make_async_copy(
                bins_v.at[pl.ds((j + 2) * BINS, BINS)],
                cnt_hbm.at[pl.ds(0, BINS)],
                osem[j + 2],
            ).wait()
        for j in range(NBUF - 2):
            pltpu.make_async_copy(
                zsp, bins_v.at[pl.ds(j * BINS, BINS)], zsem[j]).wait()

    return hist(addr_flat)


def _tc_matmul(counts2d, weight):
    M = B * C
    BM = 3328

    def body(c_ref, w_ref, o_ref):
        acc = lax.dot_general(
            c_ref[...], w_ref[...],
            dimension_numbers=(((1,), (0,)), ((), ())),
            preferred_element_type=jnp.float32,
            precision=lax.Precision.HIGHEST,
        )
        o_ref[...] = acc / float(S)

    return pl.pallas_call(
        body,
        grid=(M // BM,),
        in_specs=[
            pl.BlockSpec((BM, V), lambda i: (i, 0)),
            pl.BlockSpec((V, D), lambda i: (0, 0)),
        ],
        out_specs=pl.BlockSpec((BM, D), lambda i: (i, 0)),
        out_shape=jax.ShapeDtypeStruct((M, D), jnp.float32),
    )(counts2d, weight)


def kernel(x, weight):
    addr3 = _tc_quant(x)
    counts = _sc_hist(addr3.reshape(-1))
    out2d = _tc_matmul(counts.reshape(B * C, V), weight)
    return out2d.reshape(B, C, D)


# speedup vs baseline: 1.0875x; 1.0875x over previous
import functools

import jax
import jax.numpy as jnp
from jax import lax
from jax.experimental import pallas as pl
from jax.experimental.pallas import tpu as pltpu
from jax.experimental.pallas import tpu_sc as plsc

B = 1024
S = 50
C = 26
D = 128
V = 256
LOW = -3.0
HIGH = 3.0

NC = 2
NS = 16
NW = NC * NS
BPW = B // NW

ROW = S * C
ROWP = 1408
NRV = 82
BINS = C * V


def _tc_quant(x2):
    BM = 128

    def body(x_ref, o_ref):
        v = x_ref[...]
        t = ((v - LOW) / (HIGH - LOW)) * float(V - 1)
        q = jnp.clip(jnp.round(t), 0.0, float(V - 1))
        idx = q.astype(jnp.int32)
        ch = lax.rem(lax.broadcasted_iota(jnp.int32, (BM, ROW), 1), C)
        o_ref[:, : ROW] = ch * V + idx
        o_ref[:, ROW:] = jnp.zeros((BM, ROWP - ROW), jnp.int32)

    return pl.pallas_call(
        body,
        grid=(B // BM,),
        in_specs=[pl.BlockSpec((BM, ROW), lambda i: (i, 0))],
        out_specs=pl.BlockSpec((BM, ROWP), lambda i: (i, 0)),
        out_shape=jax.ShapeDtypeStruct((B, ROWP), jnp.int32),
    )(x2)


def _sc_hist(addr_flat):
    mesh = plsc.VectorSubcoreMesh(core_axis_name="c", subcore_axis_name="s")
    NBUF = 4

    @functools.partial(
        pl.kernel,
        out_type=jax.ShapeDtypeStruct((B * BINS,), jnp.float32),
        mesh=mesh,
        scratch_types=[
            pltpu.VMEM((BPW * ROWP,), jnp.int32),
            pltpu.VMEM((NBUF * BINS,), jnp.float32),
            pltpu.VMEM_SHARED((BINS,), jnp.float32),
            [pltpu.SemaphoreType.DMA] * NBUF,
            [pltpu.SemaphoreType.DMA] * NBUF,
        ],
        compiler_params=pltpu.CompilerParams(needs_layout_passes=False),
    )
    def hist(a_hbm, cnt_hbm, a_v, bins_v, zsp, osem, zsem):
        sid = lax.axis_index("s")
        wid = sid * NC + lax.axis_index("c")
        base_b = wid * BPW
        pltpu.sync_copy(a_hbm.at[pl.ds(base_b * ROWP, BPW * ROWP)], a_v)

        lane = lax.iota(jnp.int32, 16)
        ones = jnp.full((16,), 1.0, jnp.float32)
        zeros = jnp.zeros((16,), jnp.float32)
        tail_mask = lane < (ROW - (NRV - 1) * 16)

        def zero_body(z, c2):
            for k in range(8):
                bins_v[pl.ds(z * 128 + k * 16, 16)] = zeros
            return c2

        lax.fori_loop(0, NBUF * BINS // 128, zero_body, 0)

        @pl.when(sid == 0)
        def _publish_zeros():
            pltpu.sync_copy(bins_v.at[pl.ds(0, BINS)], zsp)

        plsc.subcore_barrier()

        def scat(bb, pbase):
            aoff = bb * ROWP
            for r in range(NRV):
                a = a_v[pl.ds(aoff + r * 16, 16)] + pbase
                if (r + 1) * 16 <= ROW:
                    plsc.addupdate_scatter(bins_v, [a], ones)
                else:
                    plsc.addupdate_scatter(bins_v, [a], ones, mask=tail_mask)

        def batch_body(i, carry):
            for q in range(NBUF):
                bb = i * NBUF + q
                pbase = q * BINS

                @pl.when(i > 0)
                def _wait_zero():
                    pltpu.make_async_copy(
                        zsp, bins_v.at[pl.ds(pbase, BINS)], zsem[q]).wait()

                scat(bb, pbase)

                pltpu.async_copy(
                    bins_v.at[pl.ds(pbase, BINS)],
                    cnt_hbm.at[pl.ds((base_b + bb) * BINS, BINS)],
                    osem[q],
                )

                r2 = (q + 2) % NBUF
                rbase = r2 * BINS

                @pl.when((i > 0) | (q >= 2))
                def _refill():
                    pltpu.make_async_copy(
                        bins_v.at[pl.ds(rbase, BINS)],
                        cnt_hbm.at[pl.ds(0, BINS)],
                        osem[r2],
                    ).wait()
                    pltpu.async_copy(
                        zsp, bins_v.at[pl.ds(rbase, BINS)], zsem[r2])
            return carry

        lax.fori_loop(0, BPW // NBUF, batch_body, 0)

        for q in range(2):
            pltpu.make_async_copy(
                bins_v.at[pl.ds((q + 2) * BINS, BINS)],
                cnt_hbm.at[pl.ds(0, BINS)],
                osem[q + 2],
            ).wait()
        for q in range(NBUF - 2):
            pltpu.make_async_copy(
                zsp, bins_v.at[pl.ds(q * BINS, BINS)], zsem[q]).wait()

    return hist(addr_flat)


def _tc_matmul(counts2d, weight):
    M = B * C
    BM = 3328

    def body(c_ref, w_ref, o_ref):
        acc = lax.dot_general(
            c_ref[...], w_ref[...],
            dimension_numbers=(((1,), (0,)), ((), ())),
            preferred_element_type=jnp.float32,
            precision=lax.Precision.HIGHEST,
        )
        o_ref[...] = acc / float(S)

    return pl.pallas_call(
        body,
        grid=(M // BM,),
        in_specs=[
            pl.BlockSpec((BM, V), lambda i: (i, 0)),
            pl.BlockSpec((V, D), lambda i: (0, 0)),
        ],
        out_specs=pl.BlockSpec((BM, D), lambda i: (i, 0)),
        out_shape=jax.ShapeDtypeStruct((M, D), jnp.float32),
    )(counts2d, weight)


def kernel(x, weight):
    addr = _tc_quant(x.reshape(B, ROW))
    counts = _sc_hist(addr.reshape(-1))
    out2d = _tc_matmul(counts.reshape(B * C, V), weight)
    return out2d.reshape(B, C, D)
